# baseline (speedup 1.0000x reference)
import functools

import jax
import jax.numpy as jnp
from jax import lax
from jax.experimental import pallas as pl
from jax.experimental.pallas import tpu as pltpu
from jax.experimental.pallas import tpu_sc as plsc

_N = 10000
_E = 320000
_D = 128
_L = 8
_G = 16

_NP = 10240
_NSC = 2
_NT = 16
_CH = 128
_NCHK = 2560
_CPT = _NCHK // (_NSC * _NT)
_NB = 4
_RPT = _NP // _NT
_BN = 1024

_sc_mesh = plsc.VectorSubcoreMesh(core_axis_name="c", subcore_axis_name="s")


_BW = 8


@functools.partial(
    pl.kernel,
    out_type=jax.ShapeDtypeStruct((_NSC, _NP, _D), jnp.float32),
    mesh=_sc_mesh,
    scratch_types=[
        pltpu.VMEM((_BW, _CH), jnp.int32),
        pltpu.VMEM((_BW, _CH), jnp.int32),
        pltpu.VMEM((_CH, _D), jnp.float32),
        pltpu.VMEM((_CH, _D), jnp.float32),
        pltpu.VMEM_SHARED((_NP, _D), jnp.float32),
        pltpu.SemaphoreType.DMA,
        pltpu.SemaphoreType.DMA,
    ],
)
def _sc_agg(src_hbm, dst_hbm, hw_hbm, zeros_hbm, out_hbm, srcv, dstv,
            b0, b1, acc, g0, g1):
    c = lax.axis_index("c")
    s = lax.axis_index("s")
    wid = c * _NT + s
    bufs = (b0, b1)
    gsems = (g0, g1)
    pltpu.sync_copy(zeros_hbm, b0)
    row0 = s * _RPT
    for j in range(_RPT // _CH):
        pltpu.sync_copy(b0, acc.at[pl.ds(row0 + j * _CH, _CH)])
    plsc.subcore_barrier()
    base = wid * _CPT

    def body(t, carry):
        pltpu.sync_copy(src_hbm.at[pl.ds(base + t * _BW, _BW)], srcv)
        pltpu.sync_copy(dst_hbm.at[pl.ds(base + t * _BW, _BW)], dstv)
        gds = [None, None]
        gds[0] = pltpu.async_copy(hw_hbm.at[srcv.at[0]], bufs[0], gsems[0])
        for j in range(_BW):
            bj = j % 2
            if j + 1 < _BW:
                bn = (j + 1) % 2
                gds[bn] = pltpu.async_copy(hw_hbm.at[srcv.at[j + 1]],
                                           bufs[bn], gsems[bn])
            gds[bj].wait()
            pltpu.sync_copy(bufs[bj], acc.at[dstv.at[j]], add=True)
        return carry

    lax.fori_loop(0, _CPT // _BW, body, 0)
    plsc.subcore_barrier()
    for j in range(_RPT // _CH):
        r = row0 + j * _CH
        pltpu.sync_copy(acc.at[pl.ds(r, _CH)], b0)
        pltpu.sync_copy(b0, out_hbm.at[c, pl.ds(r, _CH)])


def _tc_pre_body(x_ref, deg_ref, W0_ref, b0_ref, Ws0_ref, dis_ref, hw_ref):
    counts = deg_ref[0][:, 0:1] + deg_ref[1][:, 0:1]
    dis = lax.rsqrt(counts + 1.0)
    h0 = jnp.dot(x_ref[...], W0_ref[...], preferred_element_type=jnp.float32)
    h0 = h0 + b0_ref[...]
    hw = jnp.dot(h0, Ws0_ref[...], preferred_element_type=jnp.float32) * dis
    dis_ref[...] = dis
    hw_ref[...] = hw


_tc_pre = pl.pallas_call(
    _tc_pre_body,
    grid=(_NP // _BN,),
    in_specs=[
        pl.BlockSpec((_BN, _D), lambda i: (i, 0)),
        pl.BlockSpec((_NSC, _BN, _D), lambda i: (0, i, 0)),
        pl.BlockSpec((_D, _D), lambda i: (0, 0)),
        pl.BlockSpec((1, _D), lambda i: (0, 0)),
        pl.BlockSpec((_D, _D), lambda i: (0, 0)),
    ],
    out_specs=[
        pl.BlockSpec((_BN, 1), lambda i: (i, 0)),
        pl.BlockSpec((_BN, _D), lambda i: (i, 0)),
    ],
    out_shape=[
        jax.ShapeDtypeStruct((_NP, 1), jnp.float32),
        jax.ShapeDtypeStruct((_NP, _D), jnp.float32),
    ],
)


def _tc_mid_body(acc_ref, hw_ref, dis_ref, b_ref, g_ref, be_ref, Wn_ref,
                 h_o, hw_o):
    dis = dis_ref[...]
    t = (acc_ref[0] + acc_ref[1] + hw_ref[...]) * dis + b_ref[...]
    mu = jnp.mean(t, axis=-1, keepdims=True)
    d = t - mu
    var = jnp.mean(d * d, axis=-1, keepdims=True)
    tn = d * lax.rsqrt(var + 1e-5) * g_ref[...] + be_ref[...]
    h = jnp.maximum(tn, 0.0)
    h_o[...] = h
    hw_o[...] = (jnp.dot(h, Wn_ref[...], preferred_element_type=jnp.float32)
                 * dis)


_tc_mid = pl.pallas_call(
    _tc_mid_body,
    grid=(_NP // _BN,),
    in_specs=[
        pl.BlockSpec((_NSC, _BN, _D), lambda i: (0, i, 0)),
        pl.BlockSpec((_BN, _D), lambda i: (i, 0)),
        pl.BlockSpec((_BN, 1), lambda i: (i, 0)),
        pl.BlockSpec((1, _D), lambda i: (0, 0)),
        pl.BlockSpec((1, _D), lambda i: (0, 0)),
        pl.BlockSpec((1, _D), lambda i: (0, 0)),
        pl.BlockSpec((_D, _D), lambda i: (0, 0)),
    ],
    out_specs=[
        pl.BlockSpec((_BN, _D), lambda i: (i, 0)),
        pl.BlockSpec((_BN, _D), lambda i: (i, 0)),
    ],
    out_shape=[
        jax.ShapeDtypeStruct((_NP, _D), jnp.float32),
        jax.ShapeDtypeStruct((_NP, _D), jnp.float32),
    ],
)


def _tc_pool_body(h_ref, batch_ref, Wf_ref, bf_ref, out_ref, pool_ref,
                  cnt_ref):
    i = pl.program_id(0)

    @pl.when(i == 0)
    def _():
        pool_ref[...] = jnp.zeros_like(pool_ref)
        cnt_ref[...] = jnp.zeros_like(cnt_ref)

    onehot = (batch_ref[...] ==
              lax.broadcasted_iota(jnp.int32, (1, _G), 1)).astype(jnp.float32)
    pool_ref[...] += lax.dot_general(
        onehot, h_ref[...], (((0,), (0,)), ((), ())),
        preferred_element_type=jnp.float32)
    cnt_ref[...] += lax.dot_general(
        onehot, jnp.ones((_BN, _D), jnp.float32), (((0,), (0,)), ((), ())),
        preferred_element_type=jnp.float32)

    @pl.when(i == pl.num_programs(0) - 1)
    def _():
        pooled = pool_ref[...] / jnp.maximum(cnt_ref[...], 1.0)
        out_ref[...] = (jnp.dot(pooled, Wf_ref[...],
                                preferred_element_type=jnp.float32)
                        + bf_ref[...])


_tc_pool = pl.pallas_call(
    _tc_pool_body,
    grid=(_NP // _BN,),
    in_specs=[
        pl.BlockSpec((_BN, _D), lambda i: (i, 0)),
        pl.BlockSpec((_BN, 1), lambda i: (i, 0)),
        pl.BlockSpec((_D, _D), lambda i: (0, 0)),
        pl.BlockSpec((1, _D), lambda i: (0, 0)),
    ],
    out_specs=pl.BlockSpec((_G, _D), lambda i: (0, 0)),
    out_shape=jax.ShapeDtypeStruct((_G, _D), jnp.float32),
    scratch_shapes=[
        pltpu.VMEM((_G, _D), jnp.float32),
        pltpu.VMEM((_G, _D), jnp.float32),
    ],
)


def kernel(x, edge_index, batch, W0, b0, Ws, bs, gammas, betas, Wf, bf):
    padv = jnp.full((_NCHK * _CH - _E,), _NP - 1, jnp.int32)
    src2 = jnp.concatenate([edge_index[0], padv]).reshape(_NCHK, _CH)
    dst2 = jnp.concatenate([edge_index[1], padv]).reshape(_NCHK, _CH)
    xp = jnp.zeros((_NP, _D), jnp.float32).at[:_N].set(x)
    batch_p = jnp.full((_NP, 1), _G, jnp.int32).at[:_N, 0].set(batch)
    zeros_row = jnp.zeros((_CH, _D), jnp.float32)
    ones_mat = jnp.ones((_NP, _D), jnp.float32)

    deg = _sc_agg(dst2, dst2, ones_mat, zeros_row)
    dis, hw = _tc_pre(xp, deg, W0, b0[None, :], Ws[0])

    Wnext = jnp.concatenate([Ws[1:], Ws[:1]], axis=0)

    def step(carry, xs):
        hw_c, _h = carry
        Wn, b_i, g_i, be_i = xs
        accs = _sc_agg(src2, dst2, hw_c, zeros_row)
        h2, hw2 = _tc_mid(accs, hw_c, dis, b_i[None, :], g_i[None, :],
                          be_i[None, :], Wn)
        return (hw2, h2), None

    (_, h_f), _ = lax.scan(step, (hw, jnp.zeros((_NP, _D), jnp.float32)),
                           (Wnext, bs, gammas, betas))
    return _tc_pool(h_f, batch_p, Wf, bf[None, :])


# speedup vs baseline: 1.1403x; 1.1403x over previous
import functools

import jax
import jax.numpy as jnp
from jax import lax
from jax.experimental import pallas as pl
from jax.experimental.pallas import tpu as pltpu
from jax.experimental.pallas import tpu_sc as plsc

_N = 10000
_E = 320000
_D = 128
_L = 8
_G = 16

_NP = 10240
_NSC = 2
_NT = 16
_CH = 128
_NCHK = 2560
_CPT = _NCHK // (_NSC * _NT)
_NB = 4
_RPT = _NP // _NT
_BN = 1024

_sc_mesh = plsc.VectorSubcoreMesh(core_axis_name="c", subcore_axis_name="s")


_BW = 8


@functools.partial(
    pl.kernel,
    out_type=jax.ShapeDtypeStruct((_NSC, _NP, _D), jnp.float32),
    mesh=_sc_mesh,
    scratch_types=[
        pltpu.VMEM((_BW, _CH), jnp.int32),
        pltpu.VMEM((_BW, _CH), jnp.int32),
        pltpu.VMEM((_CH, _D), jnp.float32),
        pltpu.VMEM((_CH, _D), jnp.float32),
        pltpu.VMEM_SHARED((_NP, _D), jnp.float32),
        pltpu.SemaphoreType.DMA,
        pltpu.SemaphoreType.DMA,
    ],
)
def _sc_agg(src_hbm, dst_hbm, hw_hbm, zeros_hbm, out_hbm, srcv, dstv,
            b0, b1, acc, g0, g1):
    c = lax.axis_index("c")
    s = lax.axis_index("s")
    wid = c * _NT + s
    bufs = (b0, b1)
    gsems = (g0, g1)
    pltpu.sync_copy(zeros_hbm, b0)
    row0 = s * _RPT
    for j in range(_RPT // _CH):
        pltpu.sync_copy(b0, acc.at[pl.ds(row0 + j * _CH, _CH)])
    plsc.subcore_barrier()
    base = wid * _CPT

    def body(t, carry):
        pltpu.sync_copy(src_hbm.at[pl.ds(base + t * _BW, _BW)], srcv)
        pltpu.sync_copy(dst_hbm.at[pl.ds(base + t * _BW, _BW)], dstv)
        gds = [None, None]
        gds[0] = pltpu.async_copy(hw_hbm.at[srcv.at[0]], bufs[0], gsems[0])
        for j in range(_BW):
            bj = j % 2
            if j + 1 < _BW:
                bn = (j + 1) % 2
                gds[bn] = pltpu.async_copy(hw_hbm.at[srcv.at[j + 1]],
                                           bufs[bn], gsems[bn])
            gds[bj].wait()
            pltpu.sync_copy(bufs[bj], acc.at[dstv.at[j]], add=True)
        return carry

    lax.fori_loop(0, _CPT // _BW, body, 0)
    plsc.subcore_barrier()
    for j in range(_RPT // _CH):
        r = row0 + j * _CH
        pltpu.sync_copy(acc.at[pl.ds(r, _CH)], b0)
        pltpu.sync_copy(b0, out_hbm.at[c, pl.ds(r, _CH)])


def _tc_pre_body(x_ref, deg_ref, W0_ref, b0_ref, Ws0_ref, dis_ref, hw_ref):
    counts = deg_ref[0][:, 0:1] + deg_ref[1][:, 0:1]
    dis = lax.rsqrt(counts + 1.0)
    h0 = jnp.dot(x_ref[...], W0_ref[...], preferred_element_type=jnp.float32)
    h0 = h0 + b0_ref[...]
    hw = jnp.dot(h0, Ws0_ref[...], preferred_element_type=jnp.float32) * dis
    dis_ref[...] = dis
    hw_ref[...] = hw


_tc_pre = pl.pallas_call(
    _tc_pre_body,
    grid=(_NP // _BN,),
    in_specs=[
        pl.BlockSpec((_BN, _D), lambda i: (i, 0)),
        pl.BlockSpec((_NSC, _BN, _D), lambda i: (0, i, 0)),
        pl.BlockSpec((_D, _D), lambda i: (0, 0)),
        pl.BlockSpec((1, _D), lambda i: (0, 0)),
        pl.BlockSpec((_D, _D), lambda i: (0, 0)),
    ],
    out_specs=[
        pl.BlockSpec((_BN, 1), lambda i: (i, 0)),
        pl.BlockSpec((_BN, _D), lambda i: (i, 0)),
    ],
    out_shape=[
        jax.ShapeDtypeStruct((_NP, 1), jnp.float32),
        jax.ShapeDtypeStruct((_NP, _D), jnp.float32),
    ],
)


def _tc_mid_body(acc_ref, hw_ref, dis_ref, b_ref, g_ref, be_ref, Wn_ref,
                 h_o, hw_o):
    dis = dis_ref[...]
    t = (acc_ref[0] + acc_ref[1] + hw_ref[...]) * dis + b_ref[...]
    mu = jnp.mean(t, axis=-1, keepdims=True)
    d = t - mu
    var = jnp.mean(d * d, axis=-1, keepdims=True)
    tn = d * lax.rsqrt(var + 1e-5) * g_ref[...] + be_ref[...]
    h = jnp.maximum(tn, 0.0)
    h_o[...] = h
    hw_o[...] = (jnp.dot(h, Wn_ref[...], preferred_element_type=jnp.float32)
                 * dis)


_tc_mid = pl.pallas_call(
    _tc_mid_body,
    grid=(_NP // _BN,),
    in_specs=[
        pl.BlockSpec((_NSC, _BN, _D), lambda i: (0, i, 0)),
        pl.BlockSpec((_BN, _D), lambda i: (i, 0)),
        pl.BlockSpec((_BN, 1), lambda i: (i, 0)),
        pl.BlockSpec((1, _D), lambda i: (0, 0)),
        pl.BlockSpec((1, _D), lambda i: (0, 0)),
        pl.BlockSpec((1, _D), lambda i: (0, 0)),
        pl.BlockSpec((_D, _D), lambda i: (0, 0)),
    ],
    out_specs=[
        pl.BlockSpec((_BN, _D), lambda i: (i, 0)),
        pl.BlockSpec((_BN, _D), lambda i: (i, 0)),
    ],
    out_shape=[
        jax.ShapeDtypeStruct((_NP, _D), jnp.float32),
        jax.ShapeDtypeStruct((_NP, _D), jnp.float32),
    ],
)


def _tc_pool_body(h_ref, batch_ref, Wf_ref, bf_ref, out_ref, pool_ref,
                  cnt_ref):
    i = pl.program_id(0)

    @pl.when(i == 0)
    def _():
        pool_ref[...] = jnp.zeros_like(pool_ref)
        cnt_ref[...] = jnp.zeros_like(cnt_ref)

    onehot = (batch_ref[...] ==
              lax.broadcasted_iota(jnp.int32, (1, _G), 1)).astype(jnp.float32)
    pool_ref[...] += lax.dot_general(
        onehot, h_ref[...], (((0,), (0,)), ((), ())),
        preferred_element_type=jnp.float32)
    cnt_ref[...] += lax.dot_general(
        onehot, jnp.ones((_BN, _D), jnp.float32), (((0,), (0,)), ((), ())),
        preferred_element_type=jnp.float32)

    @pl.when(i == pl.num_programs(0) - 1)
    def _():
        pooled = pool_ref[...] / jnp.maximum(cnt_ref[...], 1.0)
        out_ref[...] = (jnp.dot(pooled, Wf_ref[...],
                                preferred_element_type=jnp.float32)
                        + bf_ref[...])


_tc_pool = pl.pallas_call(
    _tc_pool_body,
    grid=(_NP // _BN,),
    in_specs=[
        pl.BlockSpec((_BN, _D), lambda i: (i, 0)),
        pl.BlockSpec((_BN, 1), lambda i: (i, 0)),
        pl.BlockSpec((_D, _D), lambda i: (0, 0)),
        pl.BlockSpec((1, _D), lambda i: (0, 0)),
    ],
    out_specs=pl.BlockSpec((_G, _D), lambda i: (0, 0)),
    out_shape=jax.ShapeDtypeStruct((_G, _D), jnp.float32),
    scratch_shapes=[
        pltpu.VMEM((_G, _D), jnp.float32),
        pltpu.VMEM((_G, _D), jnp.float32),
    ],
)


def kernel(x, edge_index, batch, W0, b0, Ws, bs, gammas, betas, Wf, bf):
    padv = jnp.full((_NCHK * _CH - _E,), _NP - 1, jnp.int32)
    src2 = jnp.concatenate([edge_index[0], padv]).reshape(_NCHK, _CH)
    dst2 = jnp.concatenate([edge_index[1], padv]).reshape(_NCHK, _CH)
    xp = jnp.zeros((_NP, _D), jnp.float32).at[:_N].set(x)
    batch_p = jnp.full((_NP, 1), _G, jnp.int32).at[:_N, 0].set(batch)
    zeros_row = jnp.zeros((_CH, _D), jnp.float32)
    ones_mat = jnp.ones((_NP, _D), jnp.float32)

    deg = _sc_agg(dst2, dst2, ones_mat, zeros_row)
    dis, hw = _tc_pre(xp, deg, W0, b0[None, :], Ws[0])

    h_f = None
    for i in range(_L):
        accs = _sc_agg(src2, dst2, hw, zeros_row)
        Wn = Ws[i + 1] if i + 1 < _L else Ws[0]
        h_f, hw = _tc_mid(accs, hw, dis, bs[i][None, :], gammas[i][None, :],
                          betas[i][None, :], Wn)
    return _tc_pool(h_f, batch_p, Wf, bf[None, :])


# speedup vs baseline: 3.6798x; 3.2270x over previous
import functools

import jax
import jax.numpy as jnp
from jax import lax
from jax.experimental import pallas as pl
from jax.experimental.pallas import tpu as pltpu
from jax.experimental.pallas import tpu_sc as plsc

_N = 10000
_E = 320000
_D = 128
_L = 8
_G = 16

_NP = 10240
_NSC = 2
_NT = 16
_CH = 128
_NCHK = 2560
_CPT = _NCHK // (_NSC * _NT)
_NB = 4
_RPT = _NP // _NT
_BN = 1024

_sc_mesh = plsc.VectorSubcoreMesh(core_axis_name="c", subcore_axis_name="s")


_BW = 8


@functools.partial(
    pl.kernel,
    out_type=jax.ShapeDtypeStruct((_NSC, _NP, _D), jnp.float32),
    mesh=_sc_mesh,
    scratch_types=[
        pltpu.VMEM((_BW, _CH), jnp.int32),
        pltpu.VMEM((_BW, _CH), jnp.int32),
        pltpu.VMEM((_CH, _D), jnp.float32),
        pltpu.VMEM((_CH, _D), jnp.float32),
        pltpu.VMEM_SHARED((_NP, _D), jnp.float32),
        pltpu.SemaphoreType.DMA,
        pltpu.SemaphoreType.DMA,
    ],
)
def _sc_agg(src_hbm, dst_hbm, hw_hbm, zeros_hbm, out_hbm, srcv, dstv,
            b0, b1, acc, g0, g1):
    c = lax.axis_index("c")
    s = lax.axis_index("s")
    wid = c * _NT + s
    bufs = (b0, b1)
    gsems = (g0, g1)
    pltpu.sync_copy(zeros_hbm, b0)
    row0 = s * _RPT
    for j in range(_RPT // _CH):
        pltpu.sync_copy(b0, acc.at[pl.ds(row0 + j * _CH, _CH)])
    plsc.subcore_barrier()
    base = wid * _CPT

    def body(t, carry):
        pltpu.sync_copy(src_hbm.at[pl.ds(base + t * _BW, _BW)], srcv)
        pltpu.sync_copy(dst_hbm.at[pl.ds(base + t * _BW, _BW)], dstv)
        gds = [None, None]
        gds[0] = pltpu.async_copy(hw_hbm.at[srcv.at[0]], bufs[0], gsems[0])
        for j in range(_BW):
            bj = j % 2
            if j + 1 < _BW:
                bn = (j + 1) % 2
                gds[bn] = pltpu.async_copy(hw_hbm.at[srcv.at[j + 1]],
                                           bufs[bn], gsems[bn])
            gds[bj].wait()
            pltpu.sync_copy(bufs[bj], acc.at[dstv.at[j]], add=True)
        return carry

    lax.fori_loop(0, _CPT // _BW, body, 0)
    plsc.subcore_barrier()
    for j in range(_RPT // _CH):
        r = row0 + j * _CH
        pltpu.sync_copy(acc.at[pl.ds(r, _CH)], b0)
        pltpu.sync_copy(b0, out_hbm.at[c, pl.ds(r, _CH)])


def _tc_pre_body(x_ref, deg_ref, W0_ref, b0_ref, Ws0_ref, dis_ref, hw_ref):
    counts = deg_ref[0][:, 0:1] + deg_ref[1][:, 0:1]
    dis = lax.rsqrt(counts + 1.0)
    h0 = jnp.dot(x_ref[...], W0_ref[...], preferred_element_type=jnp.float32)
    h0 = h0 + b0_ref[...]
    hw = jnp.dot(h0, Ws0_ref[...], preferred_element_type=jnp.float32) * dis
    dis_ref[...] = dis
    hw_ref[...] = hw


_tc_pre = pl.pallas_call(
    _tc_pre_body,
    grid=(_NP // _BN,),
    in_specs=[
        pl.BlockSpec((_BN, _D), lambda i: (i, 0)),
        pl.BlockSpec((_NSC, _BN, _D), lambda i: (0, i, 0)),
        pl.BlockSpec((_D, _D), lambda i: (0, 0)),
        pl.BlockSpec((1, _D), lambda i: (0, 0)),
        pl.BlockSpec((_D, _D), lambda i: (0, 0)),
    ],
    out_specs=[
        pl.BlockSpec((_BN, 1), lambda i: (i, 0)),
        pl.BlockSpec((_BN, _D), lambda i: (i, 0)),
    ],
    out_shape=[
        jax.ShapeDtypeStruct((_NP, 1), jnp.float32),
        jax.ShapeDtypeStruct((_NP, _D), jnp.float32),
    ],
)


def _tc_mid_body(acc_ref, hw_ref, dis_ref, b_ref, g_ref, be_ref, Wn_ref,
                 h_o, hw_o):
    dis = dis_ref[...]
    t = (acc_ref[0] + acc_ref[1] + hw_ref[...]) * dis + b_ref[...]
    mu = jnp.mean(t, axis=-1, keepdims=True)
    d = t - mu
    var = jnp.mean(d * d, axis=-1, keepdims=True)
    tn = d * lax.rsqrt(var + 1e-5) * g_ref[...] + be_ref[...]
    h = jnp.maximum(tn, 0.0)
    h_o[...] = h
    hw_o[...] = (jnp.dot(h, Wn_ref[...], preferred_element_type=jnp.float32)
                 * dis)


_tc_mid = pl.pallas_call(
    _tc_mid_body,
    grid=(_NP // _BN,),
    in_specs=[
        pl.BlockSpec((_NSC, _BN, _D), lambda i: (0, i, 0)),
        pl.BlockSpec((_BN, _D), lambda i: (i, 0)),
        pl.BlockSpec((_BN, 1), lambda i: (i, 0)),
        pl.BlockSpec((1, _D), lambda i: (0, 0)),
        pl.BlockSpec((1, _D), lambda i: (0, 0)),
        pl.BlockSpec((1, _D), lambda i: (0, 0)),
        pl.BlockSpec((_D, _D), lambda i: (0, 0)),
    ],
    out_specs=[
        pl.BlockSpec((_BN, _D), lambda i: (i, 0)),
        pl.BlockSpec((_BN, _D), lambda i: (i, 0)),
    ],
    out_shape=[
        jax.ShapeDtypeStruct((_NP, _D), jnp.float32),
        jax.ShapeDtypeStruct((_NP, _D), jnp.float32),
    ],
)


def _tc_pool_body(h_ref, batch_ref, Wf_ref, bf_ref, out_ref, pool_ref,
                  cnt_ref):
    i = pl.program_id(0)

    @pl.when(i == 0)
    def _():
        pool_ref[...] = jnp.zeros_like(pool_ref)
        cnt_ref[...] = jnp.zeros_like(cnt_ref)

    onehot = (batch_ref[...] ==
              lax.broadcasted_iota(jnp.int32, (1, _G), 1)).astype(jnp.float32)
    pool_ref[...] += lax.dot_general(
        onehot, h_ref[...], (((0,), (0,)), ((), ())),
        preferred_element_type=jnp.float32)
    cnt_ref[...] += lax.dot_general(
        onehot, jnp.ones((_BN, _D), jnp.float32), (((0,), (0,)), ((), ())),
        preferred_element_type=jnp.float32)

    @pl.when(i == pl.num_programs(0) - 1)
    def _():
        pooled = pool_ref[...] / jnp.maximum(cnt_ref[...], 1.0)
        out_ref[...] = (jnp.dot(pooled, Wf_ref[...],
                                preferred_element_type=jnp.float32)
                        + bf_ref[...])


_tc_pool = pl.pallas_call(
    _tc_pool_body,
    grid=(_NP // _BN,),
    in_specs=[
        pl.BlockSpec((_BN, _D), lambda i: (i, 0)),
        pl.BlockSpec((_BN, 1), lambda i: (i, 0)),
        pl.BlockSpec((_D, _D), lambda i: (0, 0)),
        pl.BlockSpec((1, _D), lambda i: (0, 0)),
    ],
    out_specs=pl.BlockSpec((_G, _D), lambda i: (0, 0)),
    out_shape=jax.ShapeDtypeStruct((_G, _D), jnp.float32),
    scratch_shapes=[
        pltpu.VMEM((_G, _D), jnp.float32),
        pltpu.VMEM((_G, _D), jnp.float32),
    ],
)


def kernel(x, edge_index, batch, W0, b0, Ws, bs, gammas, betas, Wf, bf):
    padv = _N + jnp.arange(_NCHK * _CH - _E, dtype=jnp.int32) % (_NP - _N)
    src2 = jnp.concatenate([edge_index[0], padv]).reshape(_NCHK, _CH)
    dst2 = jnp.concatenate([edge_index[1], padv]).reshape(_NCHK, _CH)
    xp = jnp.zeros((_NP, _D), jnp.float32).at[:_N].set(x)
    batch_p = jnp.full((_NP, 1), _G, jnp.int32).at[:_N, 0].set(batch)
    zeros_row = jnp.zeros((_CH, _D), jnp.float32)
    ones_mat = jnp.ones((_NP, _D), jnp.float32)

    deg = _sc_agg(dst2, dst2, ones_mat, zeros_row)
    dis, hw = _tc_pre(xp, deg, W0, b0[None, :], Ws[0])

    h_f = None
    for i in range(_L):
        accs = _sc_agg(src2, dst2, hw, zeros_row)
        Wn = Ws[i + 1] if i + 1 < _L else Ws[0]
        h_f, hw = _tc_mid(accs, hw, dis, bs[i][None, :], gammas[i][None, :],
                          betas[i][None, :], Wn)
    return _tc_pool(h_f, batch_p, Wf, bf[None, :])


# speedup vs baseline: 3.8298x; 1.0408x over previous
import functools

import jax
import jax.numpy as jnp
from jax import lax
from jax.experimental import pallas as pl
from jax.experimental.pallas import tpu as pltpu
from jax.experimental.pallas import tpu_sc as plsc

_N = 10000
_E = 320000
_D = 128
_L = 8
_G = 16

_NP = 10240
_NSC = 2
_NT = 16
_CH = 128
_NCHK = 2560
_CPT = _NCHK // (_NSC * _NT)
_NB = 4
_RPT = _NP // _NT
_BN = 1024

_sc_mesh = plsc.VectorSubcoreMesh(core_axis_name="c", subcore_axis_name="s")


_BW = 8


@functools.partial(
    pl.kernel,
    out_type=jax.ShapeDtypeStruct((_NSC, _NP, _D), jnp.float32),
    mesh=_sc_mesh,
    scratch_types=[
        pltpu.VMEM((_BW, _CH), jnp.int32),
        pltpu.VMEM((_BW, _CH), jnp.int32),
        pltpu.VMEM((_CH, _D), jnp.float32),
        pltpu.VMEM((_CH, _D), jnp.float32),
        pltpu.VMEM_SHARED((_NP, _D), jnp.float32),
        pltpu.SemaphoreType.DMA,
        pltpu.SemaphoreType.DMA,
        pltpu.SemaphoreType.DMA,
        pltpu.SemaphoreType.DMA,
    ],
)
def _sc_agg(src_hbm, dst_hbm, hw_hbm, zeros_hbm, out_hbm, srcv, dstv,
            b0, b1, acc, g0, g1, s0, s1):
    c = lax.axis_index("c")
    s = lax.axis_index("s")
    wid = c * _NT + s
    bufs = (b0, b1)
    gsems = (g0, g1)
    ssems = (s0, s1)
    pltpu.sync_copy(zeros_hbm, b0)
    row0 = s * _RPT
    for j in range(_RPT // _CH):
        pltpu.sync_copy(b0, acc.at[pl.ds(row0 + j * _CH, _CH)])
    plsc.subcore_barrier()
    base = wid * _CPT

    def body(t, carry):
        pltpu.sync_copy(src_hbm.at[pl.ds(base + t * _BW, _BW)], srcv)
        pltpu.sync_copy(dst_hbm.at[pl.ds(base + t * _BW, _BW)], dstv)
        gds = [None, None]
        sds = [None, None]
        gds[0] = pltpu.async_copy(hw_hbm.at[srcv.at[0]], bufs[0], gsems[0])
        for j in range(_BW):
            bj = j % 2
            if j + 1 < _BW:
                bn = (j + 1) % 2
                if sds[bn] is not None:
                    sds[bn].wait()
                    sds[bn] = None
                gds[bn] = pltpu.async_copy(hw_hbm.at[srcv.at[j + 1]],
                                           bufs[bn], gsems[bn])
            gds[bj].wait()
            sds[bj] = pltpu.async_copy(bufs[bj], acc.at[dstv.at[j]],
                                       ssems[bj], add=True)
        for bj in range(2):
            if sds[bj] is not None:
                sds[bj].wait()
        return carry

    lax.fori_loop(0, _CPT // _BW, body, 0)
    plsc.subcore_barrier()
    for j in range(_RPT // _CH):
        r = row0 + j * _CH
        pltpu.sync_copy(acc.at[pl.ds(r, _CH)], b0)
        pltpu.sync_copy(b0, out_hbm.at[c, pl.ds(r, _CH)])


_CPTD = _NCHK // (_NSC * _NT)


@functools.partial(
    pl.kernel,
    out_type=jax.ShapeDtypeStruct((_NSC, _NP, _D), jnp.float32),
    mesh=_sc_mesh,
    scratch_types=[
        pltpu.VMEM((_BW, _CH), jnp.int32),
        pltpu.VMEM((_CH, _D), jnp.float32),
        pltpu.VMEM((_CH, _D), jnp.float32),
        pltpu.VMEM_SHARED((_NP, _D), jnp.float32),
        pltpu.SemaphoreType.DMA,
    ],
)
def _sc_deg(dst_hbm, ones_hbm, zeros_hbm, out_hbm, dstv, onesv, stage, acc,
            ssem):
    c = lax.axis_index("c")
    s = lax.axis_index("s")
    wid = c * _NT + s
    pltpu.sync_copy(ones_hbm, onesv)
    pltpu.sync_copy(zeros_hbm, stage)
    row0 = s * _RPT
    for j in range(_RPT // _CH):
        pltpu.sync_copy(stage, acc.at[pl.ds(row0 + j * _CH, _CH)])
    plsc.subcore_barrier()
    base = wid * _CPTD

    def body(t, carry):
        pltpu.sync_copy(dst_hbm.at[pl.ds(base + t * _BW, _BW)], dstv)
        sds = [pltpu.async_copy(onesv, acc.at[dstv.at[j]], ssem, add=True)
               for j in range(_BW)]
        for d in sds:
            d.wait()
        return carry

    lax.fori_loop(0, _CPTD // _BW, body, 0)
    plsc.subcore_barrier()
    for j in range(_RPT // _CH):
        r = row0 + j * _CH
        pltpu.sync_copy(acc.at[pl.ds(r, _CH)], stage)
        pltpu.sync_copy(stage, out_hbm.at[c, pl.ds(r, _CH)])


def _tc_pre_body(x_ref, deg_ref, W0_ref, b0_ref, Ws0_ref, dis_ref, hw_ref):
    counts = deg_ref[0][:, 0:1] + deg_ref[1][:, 0:1]
    dis = lax.rsqrt(counts + 1.0)
    h0 = jnp.dot(x_ref[...], W0_ref[...], preferred_element_type=jnp.float32)
    h0 = h0 + b0_ref[...]
    hw = jnp.dot(h0, Ws0_ref[...], preferred_element_type=jnp.float32) * dis
    dis_ref[...] = dis
    hw_ref[...] = hw


_tc_pre = pl.pallas_call(
    _tc_pre_body,
    grid=(_NP // _BN,),
    in_specs=[
        pl.BlockSpec((_BN, _D), lambda i: (i, 0)),
        pl.BlockSpec((_NSC, _BN, _D), lambda i: (0, i, 0)),
        pl.BlockSpec((_D, _D), lambda i: (0, 0)),
        pl.BlockSpec((1, _D), lambda i: (0, 0)),
        pl.BlockSpec((_D, _D), lambda i: (0, 0)),
    ],
    out_specs=[
        pl.BlockSpec((_BN, 1), lambda i: (i, 0)),
        pl.BlockSpec((_BN, _D), lambda i: (i, 0)),
    ],
    out_shape=[
        jax.ShapeDtypeStruct((_NP, 1), jnp.float32),
        jax.ShapeDtypeStruct((_NP, _D), jnp.float32),
    ],
)


def _tc_mid_body(acc_ref, hw_ref, dis_ref, b_ref, g_ref, be_ref, Wn_ref,
                 h_o, hw_o):
    dis = dis_ref[...]
    t = (acc_ref[0] + acc_ref[1] + hw_ref[...]) * dis + b_ref[...]
    mu = jnp.mean(t, axis=-1, keepdims=True)
    d = t - mu
    var = jnp.mean(d * d, axis=-1, keepdims=True)
    tn = d * lax.rsqrt(var + 1e-5) * g_ref[...] + be_ref[...]
    h = jnp.maximum(tn, 0.0)
    h_o[...] = h
    hw_o[...] = (jnp.dot(h, Wn_ref[...], preferred_element_type=jnp.float32)
                 * dis)


_tc_mid = pl.pallas_call(
    _tc_mid_body,
    grid=(_NP // _BN,),
    in_specs=[
        pl.BlockSpec((_NSC, _BN, _D), lambda i: (0, i, 0)),
        pl.BlockSpec((_BN, _D), lambda i: (i, 0)),
        pl.BlockSpec((_BN, 1), lambda i: (i, 0)),
        pl.BlockSpec((1, _D), lambda i: (0, 0)),
        pl.BlockSpec((1, _D), lambda i: (0, 0)),
        pl.BlockSpec((1, _D), lambda i: (0, 0)),
        pl.BlockSpec((_D, _D), lambda i: (0, 0)),
    ],
    out_specs=[
        pl.BlockSpec((_BN, _D), lambda i: (i, 0)),
        pl.BlockSpec((_BN, _D), lambda i: (i, 0)),
    ],
    out_shape=[
        jax.ShapeDtypeStruct((_NP, _D), jnp.float32),
        jax.ShapeDtypeStruct((_NP, _D), jnp.float32),
    ],
)


def _tc_pool_body(h_ref, batch_ref, Wf_ref, bf_ref, out_ref, pool_ref,
                  cnt_ref):
    i = pl.program_id(0)

    @pl.when(i == 0)
    def _():
        pool_ref[...] = jnp.zeros_like(pool_ref)
        cnt_ref[...] = jnp.zeros_like(cnt_ref)

    onehot = (batch_ref[...] ==
              lax.broadcasted_iota(jnp.int32, (1, _G), 1)).astype(jnp.float32)
    pool_ref[...] += lax.dot_general(
        onehot, h_ref[...], (((0,), (0,)), ((), ())),
        preferred_element_type=jnp.float32)
    cnt_ref[...] += lax.dot_general(
        onehot, jnp.ones((_BN, _D), jnp.float32), (((0,), (0,)), ((), ())),
        preferred_element_type=jnp.float32)

    @pl.when(i == pl.num_programs(0) - 1)
    def _():
        pooled = pool_ref[...] / jnp.maximum(cnt_ref[...], 1.0)
        out_ref[...] = (jnp.dot(pooled, Wf_ref[...],
                                preferred_element_type=jnp.float32)
                        + bf_ref[...])


_tc_pool = pl.pallas_call(
    _tc_pool_body,
    grid=(_NP // _BN,),
    in_specs=[
        pl.BlockSpec((_BN, _D), lambda i: (i, 0)),
        pl.BlockSpec((_BN, 1), lambda i: (i, 0)),
        pl.BlockSpec((_D, _D), lambda i: (0, 0)),
        pl.BlockSpec((1, _D), lambda i: (0, 0)),
    ],
    out_specs=pl.BlockSpec((_G, _D), lambda i: (0, 0)),
    out_shape=jax.ShapeDtypeStruct((_G, _D), jnp.float32),
    scratch_shapes=[
        pltpu.VMEM((_G, _D), jnp.float32),
        pltpu.VMEM((_G, _D), jnp.float32),
    ],
)


def kernel(x, edge_index, batch, W0, b0, Ws, bs, gammas, betas, Wf, bf):
    padv = _N + jnp.arange(_NCHK * _CH - _E, dtype=jnp.int32) % (_NP - _N)
    src2 = jnp.concatenate([edge_index[0], padv]).reshape(_NCHK, _CH)
    dst2 = jnp.concatenate([edge_index[1], padv]).reshape(_NCHK, _CH)
    xp = jnp.zeros((_NP, _D), jnp.float32).at[:_N].set(x)
    batch_p = jnp.full((_NP, 1), _G, jnp.int32).at[:_N, 0].set(batch)
    zeros_row = jnp.zeros((_CH, _D), jnp.float32)
    ones_row = jnp.ones((_CH, _D), jnp.float32)

    deg = _sc_deg(dst2, ones_row, zeros_row)
    dis, hw = _tc_pre(xp, deg, W0, b0[None, :], Ws[0])

    h_f = None
    for i in range(_L):
        accs = _sc_agg(src2, dst2, hw, zeros_row)
        Wn = Ws[i + 1] if i + 1 < _L else Ws[0]
        h_f, hw = _tc_mid(accs, hw, dis, bs[i][None, :], gammas[i][None, :],
                          betas[i][None, :], Wn)
    return _tc_pool(h_f, batch_p, Wf, bf[None, :])


# speedup vs baseline: 3.9192x; 1.0233x over previous
import functools

import jax
import jax.numpy as jnp
from jax import lax
from jax.experimental import pallas as pl
from jax.experimental.pallas import tpu as pltpu
from jax.experimental.pallas import tpu_sc as plsc

_N = 10000
_E = 320000
_D = 128
_L = 8
_G = 16

_NP = 10240
_NSC = 2
_NT = 16
_CH = 128
_NCHK = 2560
_CPT = _NCHK // (_NSC * _NT)
_NB = 4
_RPT = _NP // _NT
_BN = 1024

_sc_mesh = plsc.VectorSubcoreMesh(core_axis_name="c", subcore_axis_name="s")


_BW = 8


@functools.partial(
    pl.kernel,
    out_type=jax.ShapeDtypeStruct((_NSC, _NP, _D), jnp.float32),
    mesh=_sc_mesh,
    scratch_types=[
        pltpu.VMEM((_BW, _CH), jnp.int32),
        pltpu.VMEM((_BW, _CH), jnp.int32),
        pltpu.VMEM((_CH, _D), jnp.float32),
        pltpu.VMEM((_CH, _D), jnp.float32),
        pltpu.VMEM_SHARED((_NP, _D), jnp.float32),
        pltpu.SemaphoreType.DMA,
        pltpu.SemaphoreType.DMA,
        pltpu.SemaphoreType.DMA,
        pltpu.SemaphoreType.DMA,
    ],
)
def _sc_agg(src_hbm, dst_hbm, hw_hbm, zeros_hbm, out_hbm, srcv, dstv,
            b0, b1, acc, g0, g1, s0, s1):
    c = lax.axis_index("c")
    s = lax.axis_index("s")
    wid = c * _NT + s
    bufs = (b0, b1)
    gsems = (g0, g1)
    ssems = (s0, s1)
    pltpu.sync_copy(zeros_hbm, b0)
    row0 = s * _RPT
    zds = [pltpu.async_copy(b0, acc.at[pl.ds(row0 + j * _CH, _CH)], g0)
           for j in range(_RPT // _CH)]
    for d in zds:
        d.wait()
    plsc.subcore_barrier()
    base = wid * _CPT

    def body(t, carry):
        pltpu.sync_copy(src_hbm.at[pl.ds(base + t * _BW, _BW)], srcv)
        pltpu.sync_copy(dst_hbm.at[pl.ds(base + t * _BW, _BW)], dstv)
        gds = [None, None]
        sds = [None, None]
        gds[0] = pltpu.async_copy(hw_hbm.at[srcv.at[0]], bufs[0], gsems[0])
        for j in range(_BW):
            bj = j % 2
            if j + 1 < _BW:
                bn = (j + 1) % 2
                if sds[bn] is not None:
                    sds[bn].wait()
                    sds[bn] = None
                gds[bn] = pltpu.async_copy(hw_hbm.at[srcv.at[j + 1]],
                                           bufs[bn], gsems[bn])
            gds[bj].wait()
            sds[bj] = pltpu.async_copy(bufs[bj], acc.at[dstv.at[j]],
                                       ssems[bj], add=True)
        for bj in range(2):
            if sds[bj] is not None:
                sds[bj].wait()
        return carry

    lax.fori_loop(0, _CPT // _BW, body, 0)
    plsc.subcore_barrier()
    rds = [None, None]
    wds = [None, None]
    nf = _RPT // _CH
    rds[0] = pltpu.async_copy(acc.at[pl.ds(row0, _CH)], bufs[0], gsems[0])
    for j in range(nf):
        bj = j % 2
        if j + 1 < nf:
            bn = (j + 1) % 2
            if wds[bn] is not None:
                wds[bn].wait()
            rds[bn] = pltpu.async_copy(
                acc.at[pl.ds(row0 + (j + 1) * _CH, _CH)], bufs[bn],
                gsems[bn])
        rds[bj].wait()
        wds[bj] = pltpu.async_copy(bufs[bj],
                                   out_hbm.at[c, pl.ds(row0 + j * _CH, _CH)],
                                   ssems[bj])
    for bj in range(2):
        if wds[bj] is not None:
            wds[bj].wait()


_CPTD = _NCHK // (_NSC * _NT)


@functools.partial(
    pl.kernel,
    out_type=jax.ShapeDtypeStruct((_NSC, _NP, _D), jnp.float32),
    mesh=_sc_mesh,
    scratch_types=[
        pltpu.VMEM((_BW, _CH), jnp.int32),
        pltpu.VMEM((_CH, _D), jnp.float32),
        pltpu.VMEM((_CH, _D), jnp.float32),
        pltpu.VMEM_SHARED((_NP, _D), jnp.float32),
        pltpu.SemaphoreType.DMA,
        pltpu.SemaphoreType.DMA,
        pltpu.SemaphoreType.DMA,
        pltpu.SemaphoreType.DMA,
        pltpu.SemaphoreType.DMA,
    ],
)
def _sc_deg(dst_hbm, ones_hbm, zeros_hbm, out_hbm, dstv, onesv, stage, acc,
            ssem, r0, r1, w0, w1):
    rsems = (r0, r1)
    wsems = (w0, w1)
    c = lax.axis_index("c")
    s = lax.axis_index("s")
    wid = c * _NT + s
    pltpu.sync_copy(ones_hbm, onesv)
    pltpu.sync_copy(zeros_hbm, stage)
    row0 = s * _RPT
    zds = [pltpu.async_copy(stage, acc.at[pl.ds(row0 + j * _CH, _CH)], ssem)
           for j in range(_RPT // _CH)]
    for d in zds:
        d.wait()
    plsc.subcore_barrier()
    base = wid * _CPTD

    def body(t, carry):
        pltpu.sync_copy(dst_hbm.at[pl.ds(base + t * _BW, _BW)], dstv)
        sds = [pltpu.async_copy(onesv, acc.at[dstv.at[j]], ssem, add=True)
               for j in range(_BW)]
        for d in sds:
            d.wait()
        return carry

    lax.fori_loop(0, _CPTD // _BW, body, 0)
    plsc.subcore_barrier()
    fbufs = (stage, onesv)
    rds = [None, None]
    wds = [None, None]
    nf = _RPT // _CH
    rds[0] = pltpu.async_copy(acc.at[pl.ds(row0, _CH)], fbufs[0], rsems[0])
    for j in range(nf):
        bj = j % 2
        if j + 1 < nf:
            bn = (j + 1) % 2
            if wds[bn] is not None:
                wds[bn].wait()
            rds[bn] = pltpu.async_copy(
                acc.at[pl.ds(row0 + (j + 1) * _CH, _CH)], fbufs[bn],
                rsems[bn])
        rds[bj].wait()
        wds[bj] = pltpu.async_copy(fbufs[bj],
                                   out_hbm.at[c, pl.ds(row0 + j * _CH, _CH)],
                                   wsems[bj])
    for bj in range(2):
        if wds[bj] is not None:
            wds[bj].wait()


def _tc_pre_body(x_ref, deg_ref, W0_ref, b0_ref, Ws0_ref, dis_ref, hw_ref):
    counts = deg_ref[0][:, 0:1] + deg_ref[1][:, 0:1]
    dis = lax.rsqrt(counts + 1.0)
    h0 = jnp.dot(x_ref[...], W0_ref[...], preferred_element_type=jnp.float32)
    h0 = h0 + b0_ref[...]
    hw = jnp.dot(h0, Ws0_ref[...], preferred_element_type=jnp.float32) * dis
    dis_ref[...] = dis
    hw_ref[...] = hw


_tc_pre = pl.pallas_call(
    _tc_pre_body,
    grid=(_NP // _BN,),
    in_specs=[
        pl.BlockSpec((_BN, _D), lambda i: (i, 0)),
        pl.BlockSpec((_NSC, _BN, _D), lambda i: (0, i, 0)),
        pl.BlockSpec((_D, _D), lambda i: (0, 0)),
        pl.BlockSpec((1, _D), lambda i: (0, 0)),
        pl.BlockSpec((_D, _D), lambda i: (0, 0)),
    ],
    out_specs=[
        pl.BlockSpec((_BN, 1), lambda i: (i, 0)),
        pl.BlockSpec((_BN, _D), lambda i: (i, 0)),
    ],
    out_shape=[
        jax.ShapeDtypeStruct((_NP, 1), jnp.float32),
        jax.ShapeDtypeStruct((_NP, _D), jnp.float32),
    ],
)


def _tc_mid_body(acc_ref, hw_ref, dis_ref, b_ref, g_ref, be_ref, Wn_ref,
                 hw_o):
    dis = dis_ref[...]
    t = (acc_ref[0] + acc_ref[1] + hw_ref[...]) * dis + b_ref[...]
    mu = jnp.mean(t, axis=-1, keepdims=True)
    d = t - mu
    var = jnp.mean(d * d, axis=-1, keepdims=True)
    tn = d * lax.rsqrt(var + 1e-5) * g_ref[...] + be_ref[...]
    h = jnp.maximum(tn, 0.0)
    hw_o[...] = (jnp.dot(h, Wn_ref[...], preferred_element_type=jnp.float32)
                 * dis)


_tc_mid = pl.pallas_call(
    _tc_mid_body,
    grid=(_NP // _BN,),
    in_specs=[
        pl.BlockSpec((_NSC, _BN, _D), lambda i: (0, i, 0)),
        pl.BlockSpec((_BN, _D), lambda i: (i, 0)),
        pl.BlockSpec((_BN, 1), lambda i: (i, 0)),
        pl.BlockSpec((1, _D), lambda i: (0, 0)),
        pl.BlockSpec((1, _D), lambda i: (0, 0)),
        pl.BlockSpec((1, _D), lambda i: (0, 0)),
        pl.BlockSpec((_D, _D), lambda i: (0, 0)),
    ],
    out_specs=pl.BlockSpec((_BN, _D), lambda i: (i, 0)),
    out_shape=jax.ShapeDtypeStruct((_NP, _D), jnp.float32),
)


def _tc_last_body(acc_ref, hw_ref, dis_ref, b_ref, g_ref, be_ref, batch_ref,
                  Wf_ref, bf_ref, out_ref, pool_ref, cnt_ref):
    i = pl.program_id(0)

    @pl.when(i == 0)
    def _():
        pool_ref[...] = jnp.zeros_like(pool_ref)
        cnt_ref[...] = jnp.zeros_like(cnt_ref)

    dis = dis_ref[...]
    t = (acc_ref[0] + acc_ref[1] + hw_ref[...]) * dis + b_ref[...]
    mu = jnp.mean(t, axis=-1, keepdims=True)
    d = t - mu
    var = jnp.mean(d * d, axis=-1, keepdims=True)
    tn = d * lax.rsqrt(var + 1e-5) * g_ref[...] + be_ref[...]
    h = jnp.maximum(tn, 0.0)
    onehot = (batch_ref[...] ==
              lax.broadcasted_iota(jnp.int32, (1, _G), 1)).astype(jnp.float32)
    pool_ref[...] += lax.dot_general(
        onehot, h, (((0,), (0,)), ((), ())),
        preferred_element_type=jnp.float32)
    cnt_ref[...] += lax.dot_general(
        onehot, jnp.ones((_BN, _D), jnp.float32), (((0,), (0,)), ((), ())),
        preferred_element_type=jnp.float32)

    @pl.when(i == pl.num_programs(0) - 1)
    def _():
        pooled = pool_ref[...] / jnp.maximum(cnt_ref[...], 1.0)
        out_ref[...] = (jnp.dot(pooled, Wf_ref[...],
                                preferred_element_type=jnp.float32)
                        + bf_ref[...])


_tc_last = pl.pallas_call(
    _tc_last_body,
    grid=(_NP // _BN,),
    in_specs=[
        pl.BlockSpec((_NSC, _BN, _D), lambda i: (0, i, 0)),
        pl.BlockSpec((_BN, _D), lambda i: (i, 0)),
        pl.BlockSpec((_BN, 1), lambda i: (i, 0)),
        pl.BlockSpec((1, _D), lambda i: (0, 0)),
        pl.BlockSpec((1, _D), lambda i: (0, 0)),
        pl.BlockSpec((1, _D), lambda i: (0, 0)),
        pl.BlockSpec((_BN, 1), lambda i: (i, 0)),
        pl.BlockSpec((_D, _D), lambda i: (0, 0)),
        pl.BlockSpec((1, _D), lambda i: (0, 0)),
    ],
    out_specs=pl.BlockSpec((_G, _D), lambda i: (0, 0)),
    out_shape=jax.ShapeDtypeStruct((_G, _D), jnp.float32),
    scratch_shapes=[
        pltpu.VMEM((_G, _D), jnp.float32),
        pltpu.VMEM((_G, _D), jnp.float32),
    ],
)


def kernel(x, edge_index, batch, W0, b0, Ws, bs, gammas, betas, Wf, bf):
    padv = _N + jnp.arange(_NCHK * _CH - _E, dtype=jnp.int32) % (_NP - _N)
    src2 = jnp.concatenate([edge_index[0], padv]).reshape(_NCHK, _CH)
    dst2 = jnp.concatenate([edge_index[1], padv]).reshape(_NCHK, _CH)
    xp = jnp.zeros((_NP, _D), jnp.float32).at[:_N].set(x)
    batch_p = jnp.full((_NP, 1), _G, jnp.int32).at[:_N, 0].set(batch)
    zeros_row = jnp.zeros((_CH, _D), jnp.float32)
    ones_row = jnp.ones((_CH, _D), jnp.float32)

    deg = _sc_deg(dst2, ones_row, zeros_row)
    dis, hw = _tc_pre(xp, deg, W0, b0[None, :], Ws[0])

    for i in range(_L - 1):
        accs = _sc_agg(src2, dst2, hw, zeros_row)
        hw = _tc_mid(accs, hw, dis, bs[i][None, :], gammas[i][None, :],
                     betas[i][None, :], Ws[i + 1])
    accs = _sc_agg(src2, dst2, hw, zeros_row)
    i = _L - 1
    return _tc_last(accs, hw, dis, bs[i][None, :], gammas[i][None, :],
                    betas[i][None, :], batch_p, Wf, bf[None, :])


# speedup vs baseline: 4.2420x; 1.0824x over previous
import functools

import jax
import jax.numpy as jnp
from jax import lax
from jax.experimental import pallas as pl
from jax.experimental.pallas import tpu as pltpu
from jax.experimental.pallas import tpu_sc as plsc

_N = 10000
_E = 320000
_D = 128
_L = 8
_G = 16

_NP = 10240
_NSC = 2
_NT = 16
_CH = 128
_NCHK = 2560
_CPT = _NCHK // (_NSC * _NT)
_NB = 4
_RPT = _NP // _NT
_BN = 1024

_sc_mesh = plsc.VectorSubcoreMesh(core_axis_name="c", subcore_axis_name="s")


_BW = 16


@functools.partial(
    pl.kernel,
    out_type=jax.ShapeDtypeStruct((_NSC, _NP, _D), jnp.float32),
    mesh=_sc_mesh,
    scratch_types=[
        pltpu.VMEM((_BW, _CH), jnp.int32),
        pltpu.VMEM((_BW, _CH), jnp.int32),
        pltpu.VMEM((_CH, _D), jnp.float32),
        pltpu.VMEM((_CH, _D), jnp.float32),
        pltpu.VMEM_SHARED((_NP, _D), jnp.float32),
        pltpu.SemaphoreType.DMA,
        pltpu.SemaphoreType.DMA,
        pltpu.SemaphoreType.DMA,
        pltpu.SemaphoreType.DMA,
    ],
)
def _sc_agg(src_hbm, dst_hbm, hw_hbm, zeros_hbm, out_hbm, srcv, dstv,
            b0, b1, acc, g0, g1, s0, s1):
    c = lax.axis_index("c")
    s = lax.axis_index("s")
    wid = c * _NT + s
    bufs = (b0, b1)
    gsems = (g0, g1)
    ssems = (s0, s1)
    pltpu.sync_copy(zeros_hbm, b0)
    row0 = s * _RPT
    zds = [pltpu.async_copy(b0, acc.at[pl.ds(row0 + j * _CH, _CH)], g0)
           for j in range(_RPT // _CH)]
    for d in zds:
        d.wait()
    plsc.subcore_barrier()
    base = wid * _CPT

    def body(t, carry):
        pltpu.sync_copy(src_hbm.at[pl.ds(base + t * _BW, _BW)], srcv)
        pltpu.sync_copy(dst_hbm.at[pl.ds(base + t * _BW, _BW)], dstv)
        gds = [None, None]
        sds = [None, None]
        gds[0] = pltpu.async_copy(hw_hbm.at[srcv.at[0]], bufs[0], gsems[0])
        for j in range(_BW):
            bj = j % 2
            if j + 1 < _BW:
                bn = (j + 1) % 2
                if sds[bn] is not None:
                    sds[bn].wait()
                    sds[bn] = None
                gds[bn] = pltpu.async_copy(hw_hbm.at[srcv.at[j + 1]],
                                           bufs[bn], gsems[bn])
            gds[bj].wait()
            sds[bj] = pltpu.async_copy(bufs[bj], acc.at[dstv.at[j]],
                                       ssems[bj], add=True)
        for bj in range(2):
            if sds[bj] is not None:
                sds[bj].wait()
        return carry

    lax.fori_loop(0, _CPT // _BW, body, 0)
    plsc.subcore_barrier()
    rds = [None, None]
    wds = [None, None]
    nf = _RPT // _CH
    rds[0] = pltpu.async_copy(acc.at[pl.ds(row0, _CH)], bufs[0], gsems[0])
    for j in range(nf):
        bj = j % 2
        if j + 1 < nf:
            bn = (j + 1) % 2
            if wds[bn] is not None:
                wds[bn].wait()
            rds[bn] = pltpu.async_copy(
                acc.at[pl.ds(row0 + (j + 1) * _CH, _CH)], bufs[bn],
                gsems[bn])
        rds[bj].wait()
        wds[bj] = pltpu.async_copy(bufs[bj],
                                   out_hbm.at[c, pl.ds(row0 + j * _CH, _CH)],
                                   ssems[bj])
    for bj in range(2):
        if wds[bj] is not None:
            wds[bj].wait()


_CPTD = _NCHK // (_NSC * _NT)


@functools.partial(
    pl.kernel,
    out_type=jax.ShapeDtypeStruct((_NSC, _NP, _D), jnp.float32),
    mesh=_sc_mesh,
    scratch_types=[
        pltpu.VMEM((_BW, _CH), jnp.int32),
        pltpu.VMEM((_CH, _D), jnp.float32),
        pltpu.VMEM((_CH, _D), jnp.float32),
        pltpu.VMEM_SHARED((_NP, _D), jnp.float32),
        pltpu.SemaphoreType.DMA,
        pltpu.SemaphoreType.DMA,
        pltpu.SemaphoreType.DMA,
        pltpu.SemaphoreType.DMA,
        pltpu.SemaphoreType.DMA,
    ],
)
def _sc_deg(dst_hbm, ones_hbm, zeros_hbm, out_hbm, dstv, onesv, stage, acc,
            ssem, r0, r1, w0, w1):
    rsems = (r0, r1)
    wsems = (w0, w1)
    c = lax.axis_index("c")
    s = lax.axis_index("s")
    wid = c * _NT + s
    pltpu.sync_copy(ones_hbm, onesv)
    pltpu.sync_copy(zeros_hbm, stage)
    row0 = s * _RPT
    zds = [pltpu.async_copy(stage, acc.at[pl.ds(row0 + j * _CH, _CH)], ssem)
           for j in range(_RPT // _CH)]
    for d in zds:
        d.wait()
    plsc.subcore_barrier()
    base = wid * _CPTD

    def body(t, carry):
        pltpu.sync_copy(dst_hbm.at[pl.ds(base + t * _BW, _BW)], dstv)
        sds = [pltpu.async_copy(onesv, acc.at[dstv.at[j]], ssem, add=True)
               for j in range(_BW)]
        for d in sds:
            d.wait()
        return carry

    lax.fori_loop(0, _CPTD // _BW, body, 0)
    plsc.subcore_barrier()
    fbufs = (stage, onesv)
    rds = [None, None]
    wds = [None, None]
    nf = _RPT // _CH
    rds[0] = pltpu.async_copy(acc.at[pl.ds(row0, _CH)], fbufs[0], rsems[0])
    for j in range(nf):
        bj = j % 2
        if j + 1 < nf:
            bn = (j + 1) % 2
            if wds[bn] is not None:
                wds[bn].wait()
            rds[bn] = pltpu.async_copy(
                acc.at[pl.ds(row0 + (j + 1) * _CH, _CH)], fbufs[bn],
                rsems[bn])
        rds[bj].wait()
        wds[bj] = pltpu.async_copy(fbufs[bj],
                                   out_hbm.at[c, pl.ds(row0 + j * _CH, _CH)],
                                   wsems[bj])
    for bj in range(2):
        if wds[bj] is not None:
            wds[bj].wait()


def _tc_pre_body(x_ref, deg_ref, W0_ref, b0_ref, Ws0_ref, dis_ref, hw_ref):
    counts = deg_ref[0][:, 0:1] + deg_ref[1][:, 0:1]
    dis = lax.rsqrt(counts + 1.0)
    h0 = jnp.dot(x_ref[...], W0_ref[...], preferred_element_type=jnp.float32)
    h0 = h0 + b0_ref[...]
    hw = jnp.dot(h0, Ws0_ref[...], preferred_element_type=jnp.float32) * dis
    dis_ref[...] = dis
    hw_ref[...] = hw


_tc_pre = pl.pallas_call(
    _tc_pre_body,
    grid=(_NP // _BN,),
    in_specs=[
        pl.BlockSpec((_BN, _D), lambda i: (i, 0)),
        pl.BlockSpec((_NSC, _BN, _D), lambda i: (0, i, 0)),
        pl.BlockSpec((_D, _D), lambda i: (0, 0)),
        pl.BlockSpec((1, _D), lambda i: (0, 0)),
        pl.BlockSpec((_D, _D), lambda i: (0, 0)),
    ],
    out_specs=[
        pl.BlockSpec((_BN, 1), lambda i: (i, 0)),
        pl.BlockSpec((_BN, _D), lambda i: (i, 0)),
    ],
    out_shape=[
        jax.ShapeDtypeStruct((_NP, 1), jnp.float32),
        jax.ShapeDtypeStruct((_NP, _D), jnp.float32),
    ],
)


def _tc_mid_body(acc_ref, hw_ref, dis_ref, b_ref, g_ref, be_ref, Wn_ref,
                 hw_o):
    dis = dis_ref[...]
    t = (acc_ref[0] + acc_ref[1] + hw_ref[...]) * dis + b_ref[...]
    mu = jnp.mean(t, axis=-1, keepdims=True)
    d = t - mu
    var = jnp.mean(d * d, axis=-1, keepdims=True)
    tn = d * lax.rsqrt(var + 1e-5) * g_ref[...] + be_ref[...]
    h = jnp.maximum(tn, 0.0)
    hw_o[...] = (jnp.dot(h, Wn_ref[...], preferred_element_type=jnp.float32)
                 * dis)


_tc_mid = pl.pallas_call(
    _tc_mid_body,
    grid=(_NP // _BN,),
    in_specs=[
        pl.BlockSpec((_NSC, _BN, _D), lambda i: (0, i, 0)),
        pl.BlockSpec((_BN, _D), lambda i: (i, 0)),
        pl.BlockSpec((_BN, 1), lambda i: (i, 0)),
        pl.BlockSpec((1, _D), lambda i: (0, 0)),
        pl.BlockSpec((1, _D), lambda i: (0, 0)),
        pl.BlockSpec((1, _D), lambda i: (0, 0)),
        pl.BlockSpec((_D, _D), lambda i: (0, 0)),
    ],
    out_specs=pl.BlockSpec((_BN, _D), lambda i: (i, 0)),
    out_shape=jax.ShapeDtypeStruct((_NP, _D), jnp.float32),
)


def _tc_last_body(acc_ref, hw_ref, dis_ref, b_ref, g_ref, be_ref, batch_ref,
                  Wf_ref, bf_ref, out_ref, pool_ref, cnt_ref):
    i = pl.program_id(0)

    @pl.when(i == 0)
    def _():
        pool_ref[...] = jnp.zeros_like(pool_ref)
        cnt_ref[...] = jnp.zeros_like(cnt_ref)

    dis = dis_ref[...]
    t = (acc_ref[0] + acc_ref[1] + hw_ref[...]) * dis + b_ref[...]
    mu = jnp.mean(t, axis=-1, keepdims=True)
    d = t - mu
    var = jnp.mean(d * d, axis=-1, keepdims=True)
    tn = d * lax.rsqrt(var + 1e-5) * g_ref[...] + be_ref[...]
    h = jnp.maximum(tn, 0.0)
    onehot = (batch_ref[...] ==
              lax.broadcasted_iota(jnp.int32, (1, _G), 1)).astype(jnp.float32)
    pool_ref[...] += lax.dot_general(
        onehot, h, (((0,), (0,)), ((), ())),
        preferred_element_type=jnp.float32)
    cnt_ref[...] += lax.dot_general(
        onehot, jnp.ones((_BN, _D), jnp.float32), (((0,), (0,)), ((), ())),
        preferred_element_type=jnp.float32)

    @pl.when(i == pl.num_programs(0) - 1)
    def _():
        pooled = pool_ref[...] / jnp.maximum(cnt_ref[...], 1.0)
        out_ref[...] = (jnp.dot(pooled, Wf_ref[...],
                                preferred_element_type=jnp.float32)
                        + bf_ref[...])


_tc_last = pl.pallas_call(
    _tc_last_body,
    grid=(_NP // _BN,),
    in_specs=[
        pl.BlockSpec((_NSC, _BN, _D), lambda i: (0, i, 0)),
        pl.BlockSpec((_BN, _D), lambda i: (i, 0)),
        pl.BlockSpec((_BN, 1), lambda i: (i, 0)),
        pl.BlockSpec((1, _D), lambda i: (0, 0)),
        pl.BlockSpec((1, _D), lambda i: (0, 0)),
        pl.BlockSpec((1, _D), lambda i: (0, 0)),
        pl.BlockSpec((_BN, 1), lambda i: (i, 0)),
        pl.BlockSpec((_D, _D), lambda i: (0, 0)),
        pl.BlockSpec((1, _D), lambda i: (0, 0)),
    ],
    out_specs=pl.BlockSpec((_G, _D), lambda i: (0, 0)),
    out_shape=jax.ShapeDtypeStruct((_G, _D), jnp.float32),
    scratch_shapes=[
        pltpu.VMEM((_G, _D), jnp.float32),
        pltpu.VMEM((_G, _D), jnp.float32),
    ],
)


def kernel(x, edge_index, batch, W0, b0, Ws, bs, gammas, betas, Wf, bf):
    padv = _N + jnp.arange(_NCHK * _CH - _E, dtype=jnp.int32) % (_NP - _N)
    src2 = jnp.concatenate([edge_index[0], padv]).reshape(_NCHK, _CH)
    dst2 = jnp.concatenate([edge_index[1], padv]).reshape(_NCHK, _CH)
    xp = jnp.zeros((_NP, _D), jnp.float32).at[:_N].set(x)
    batch_p = jnp.full((_NP, 1), _G, jnp.int32).at[:_N, 0].set(batch)
    zeros_row = jnp.zeros((_CH, _D), jnp.float32)
    ones_row = jnp.ones((_CH, _D), jnp.float32)

    deg = _sc_deg(dst2, ones_row, zeros_row)
    dis, hw = _tc_pre(xp, deg, W0, b0[None, :], Ws[0])

    for i in range(_L - 1):
        accs = _sc_agg(src2, dst2, hw, zeros_row)
        hw = _tc_mid(accs, hw, dis, bs[i][None, :], gammas[i][None, :],
                     betas[i][None, :], Ws[i + 1])
    accs = _sc_agg(src2, dst2, hw, zeros_row)
    i = _L - 1
    return _tc_last(accs, hw, dis, bs[i][None, :], gammas[i][None, :],
                    betas[i][None, :], batch_p, Wf, bf[None, :])


# speedup vs baseline: 4.5026x; 1.0615x over previous
import functools

import jax
import jax.numpy as jnp
from jax import lax
from jax.experimental import pallas as pl
from jax.experimental.pallas import tpu as pltpu
from jax.experimental.pallas import tpu_sc as plsc

_N = 10000
_E = 320000
_D = 128
_L = 8
_G = 16

_NP = 10240
_NSC = 2
_NT = 16
_CH = 128
_NCHK = 2560
_CPT = _NCHK // (_NSC * _NT)
_NB = 4
_RPT = _NP // _NT
_BN = 1024

_sc_mesh = plsc.VectorSubcoreMesh(core_axis_name="c", subcore_axis_name="s")


_BW = 40


@functools.partial(
    pl.kernel,
    out_type=jax.ShapeDtypeStruct((_NSC, _NP, _D), jnp.float32),
    mesh=_sc_mesh,
    scratch_types=[
        pltpu.VMEM((_BW, _CH), jnp.int32),
        pltpu.VMEM((_BW, _CH), jnp.int32),
        pltpu.VMEM((_CH, _D), jnp.float32),
        pltpu.VMEM((_CH, _D), jnp.float32),
        pltpu.VMEM_SHARED((_NP, _D), jnp.float32),
        pltpu.SemaphoreType.DMA,
        pltpu.SemaphoreType.DMA,
        pltpu.SemaphoreType.DMA,
        pltpu.SemaphoreType.DMA,
    ],
)
def _sc_agg(src_hbm, dst_hbm, hw_hbm, zeros_hbm, out_hbm, srcv, dstv,
            b0, b1, acc, g0, g1, s0, s1):
    c = lax.axis_index("c")
    s = lax.axis_index("s")
    wid = c * _NT + s
    bufs = (b0, b1)
    gsems = (g0, g1)
    ssems = (s0, s1)
    pltpu.sync_copy(zeros_hbm, b0)
    row0 = s * _RPT
    zds = [pltpu.async_copy(b0, acc.at[pl.ds(row0 + j * _CH, _CH)], g0)
           for j in range(_RPT // _CH)]
    for d in zds:
        d.wait()
    plsc.subcore_barrier()
    base = wid * _CPT

    def body(t, carry):
        pltpu.sync_copy(src_hbm.at[pl.ds(base + t * _BW, _BW)], srcv)
        pltpu.sync_copy(dst_hbm.at[pl.ds(base + t * _BW, _BW)], dstv)
        gds = [None, None]
        sds = [None, None]
        gds[0] = pltpu.async_copy(hw_hbm.at[srcv.at[0]], bufs[0], gsems[0])
        for j in range(_BW):
            bj = j % 2
            if j + 1 < _BW:
                bn = (j + 1) % 2
                if sds[bn] is not None:
                    sds[bn].wait()
                    sds[bn] = None
                gds[bn] = pltpu.async_copy(hw_hbm.at[srcv.at[j + 1]],
                                           bufs[bn], gsems[bn])
            gds[bj].wait()
            sds[bj] = pltpu.async_copy(bufs[bj], acc.at[dstv.at[j]],
                                       ssems[bj], add=True)
        for bj in range(2):
            if sds[bj] is not None:
                sds[bj].wait()
        return carry

    lax.fori_loop(0, _CPT // _BW, body, 0)
    plsc.subcore_barrier()
    rds = [None, None]
    wds = [None, None]
    nf = _RPT // _CH
    rds[0] = pltpu.async_copy(acc.at[pl.ds(row0, _CH)], bufs[0], gsems[0])
    for j in range(nf):
        bj = j % 2
        if j + 1 < nf:
            bn = (j + 1) % 2
            if wds[bn] is not None:
                wds[bn].wait()
            rds[bn] = pltpu.async_copy(
                acc.at[pl.ds(row0 + (j + 1) * _CH, _CH)], bufs[bn],
                gsems[bn])
        rds[bj].wait()
        wds[bj] = pltpu.async_copy(bufs[bj],
                                   out_hbm.at[c, pl.ds(row0 + j * _CH, _CH)],
                                   ssems[bj])
    for bj in range(2):
        if wds[bj] is not None:
            wds[bj].wait()


_CPTD = _NCHK // (_NSC * _NT)


@functools.partial(
    pl.kernel,
    out_type=jax.ShapeDtypeStruct((_NSC, _NP, _D), jnp.float32),
    mesh=_sc_mesh,
    scratch_types=[
        pltpu.VMEM((_BW, _CH), jnp.int32),
        pltpu.VMEM((_CH, _D), jnp.float32),
        pltpu.VMEM((_CH, _D), jnp.float32),
        pltpu.VMEM_SHARED((_NP, _D), jnp.float32),
        pltpu.SemaphoreType.DMA,
        pltpu.SemaphoreType.DMA,
        pltpu.SemaphoreType.DMA,
        pltpu.SemaphoreType.DMA,
        pltpu.SemaphoreType.DMA,
    ],
)
def _sc_deg(dst_hbm, ones_hbm, zeros_hbm, out_hbm, dstv, onesv, stage, acc,
            ssem, r0, r1, w0, w1):
    rsems = (r0, r1)
    wsems = (w0, w1)
    c = lax.axis_index("c")
    s = lax.axis_index("s")
    wid = c * _NT + s
    pltpu.sync_copy(ones_hbm, onesv)
    pltpu.sync_copy(zeros_hbm, stage)
    row0 = s * _RPT
    zds = [pltpu.async_copy(stage, acc.at[pl.ds(row0 + j * _CH, _CH)], ssem)
           for j in range(_RPT // _CH)]
    for d in zds:
        d.wait()
    plsc.subcore_barrier()
    base = wid * _CPTD

    def body(t, carry):
        pltpu.sync_copy(dst_hbm.at[pl.ds(base + t * _BW, _BW)], dstv)
        sds = [pltpu.async_copy(onesv, acc.at[dstv.at[j]], ssem, add=True)
               for j in range(_BW)]
        for d in sds:
            d.wait()
        return carry

    lax.fori_loop(0, _CPTD // _BW, body, 0)
    plsc.subcore_barrier()
    fbufs = (stage, onesv)
    rds = [None, None]
    wds = [None, None]
    nf = _RPT // _CH
    rds[0] = pltpu.async_copy(acc.at[pl.ds(row0, _CH)], fbufs[0], rsems[0])
    for j in range(nf):
        bj = j % 2
        if j + 1 < nf:
            bn = (j + 1) % 2
            if wds[bn] is not None:
                wds[bn].wait()
            rds[bn] = pltpu.async_copy(
                acc.at[pl.ds(row0 + (j + 1) * _CH, _CH)], fbufs[bn],
                rsems[bn])
        rds[bj].wait()
        wds[bj] = pltpu.async_copy(fbufs[bj],
                                   out_hbm.at[c, pl.ds(row0 + j * _CH, _CH)],
                                   wsems[bj])
    for bj in range(2):
        if wds[bj] is not None:
            wds[bj].wait()


def _tc_pre_body(x_ref, deg_ref, W0_ref, b0_ref, Ws0_ref, dis_ref, hw_ref):
    counts = deg_ref[0][:, 0:1] + deg_ref[1][:, 0:1]
    dis = lax.rsqrt(counts + 1.0)
    h0 = jnp.dot(x_ref[...], W0_ref[...], preferred_element_type=jnp.float32)
    h0 = h0 + b0_ref[...]
    hw = jnp.dot(h0, Ws0_ref[...], preferred_element_type=jnp.float32) * dis
    dis_ref[...] = dis
    hw_ref[...] = hw


_tc_pre = pl.pallas_call(
    _tc_pre_body,
    grid=(_NP // _BN,),
    in_specs=[
        pl.BlockSpec((_BN, _D), lambda i: (i, 0)),
        pl.BlockSpec((_NSC, _BN, _D), lambda i: (0, i, 0)),
        pl.BlockSpec((_D, _D), lambda i: (0, 0)),
        pl.BlockSpec((1, _D), lambda i: (0, 0)),
        pl.BlockSpec((_D, _D), lambda i: (0, 0)),
    ],
    out_specs=[
        pl.BlockSpec((_BN, 1), lambda i: (i, 0)),
        pl.BlockSpec((_BN, _D), lambda i: (i, 0)),
    ],
    out_shape=[
        jax.ShapeDtypeStruct((_NP, 1), jnp.float32),
        jax.ShapeDtypeStruct((_NP, _D), jnp.float32),
    ],
)


def _tc_mid_body(acc_ref, hw_ref, dis_ref, b_ref, g_ref, be_ref, Wn_ref,
                 hw_o):
    dis = dis_ref[...]
    t = (acc_ref[0] + acc_ref[1] + hw_ref[...]) * dis + b_ref[...]
    mu = jnp.mean(t, axis=-1, keepdims=True)
    d = t - mu
    var = jnp.mean(d * d, axis=-1, keepdims=True)
    tn = d * lax.rsqrt(var + 1e-5) * g_ref[...] + be_ref[...]
    h = jnp.maximum(tn, 0.0)
    hw_o[...] = (jnp.dot(h, Wn_ref[...], preferred_element_type=jnp.float32)
                 * dis)


_tc_mid = pl.pallas_call(
    _tc_mid_body,
    grid=(_NP // _BN,),
    in_specs=[
        pl.BlockSpec((_NSC, _BN, _D), lambda i: (0, i, 0)),
        pl.BlockSpec((_BN, _D), lambda i: (i, 0)),
        pl.BlockSpec((_BN, 1), lambda i: (i, 0)),
        pl.BlockSpec((1, _D), lambda i: (0, 0)),
        pl.BlockSpec((1, _D), lambda i: (0, 0)),
        pl.BlockSpec((1, _D), lambda i: (0, 0)),
        pl.BlockSpec((_D, _D), lambda i: (0, 0)),
    ],
    out_specs=pl.BlockSpec((_BN, _D), lambda i: (i, 0)),
    out_shape=jax.ShapeDtypeStruct((_NP, _D), jnp.float32),
)


def _tc_last_body(acc_ref, hw_ref, dis_ref, b_ref, g_ref, be_ref, batch_ref,
                  Wf_ref, bf_ref, out_ref, pool_ref, cnt_ref):
    i = pl.program_id(0)

    @pl.when(i == 0)
    def _():
        pool_ref[...] = jnp.zeros_like(pool_ref)
        cnt_ref[...] = jnp.zeros_like(cnt_ref)

    dis = dis_ref[...]
    t = (acc_ref[0] + acc_ref[1] + hw_ref[...]) * dis + b_ref[...]
    mu = jnp.mean(t, axis=-1, keepdims=True)
    d = t - mu
    var = jnp.mean(d * d, axis=-1, keepdims=True)
    tn = d * lax.rsqrt(var + 1e-5) * g_ref[...] + be_ref[...]
    h = jnp.maximum(tn, 0.0)
    onehot = (batch_ref[...] ==
              lax.broadcasted_iota(jnp.int32, (1, _G), 1)).astype(jnp.float32)
    pool_ref[...] += lax.dot_general(
        onehot, h, (((0,), (0,)), ((), ())),
        preferred_element_type=jnp.float32)
    cnt_ref[...] += lax.dot_general(
        onehot, jnp.ones((_BN, _D), jnp.float32), (((0,), (0,)), ((), ())),
        preferred_element_type=jnp.float32)

    @pl.when(i == pl.num_programs(0) - 1)
    def _():
        pooled = pool_ref[...] / jnp.maximum(cnt_ref[...], 1.0)
        out_ref[...] = (jnp.dot(pooled, Wf_ref[...],
                                preferred_element_type=jnp.float32)
                        + bf_ref[...])


_tc_last = pl.pallas_call(
    _tc_last_body,
    grid=(_NP // _BN,),
    in_specs=[
        pl.BlockSpec((_NSC, _BN, _D), lambda i: (0, i, 0)),
        pl.BlockSpec((_BN, _D), lambda i: (i, 0)),
        pl.BlockSpec((_BN, 1), lambda i: (i, 0)),
        pl.BlockSpec((1, _D), lambda i: (0, 0)),
        pl.BlockSpec((1, _D), lambda i: (0, 0)),
        pl.BlockSpec((1, _D), lambda i: (0, 0)),
        pl.BlockSpec((_BN, 1), lambda i: (i, 0)),
        pl.BlockSpec((_D, _D), lambda i: (0, 0)),
        pl.BlockSpec((1, _D), lambda i: (0, 0)),
    ],
    out_specs=pl.BlockSpec((_G, _D), lambda i: (0, 0)),
    out_shape=jax.ShapeDtypeStruct((_G, _D), jnp.float32),
    scratch_shapes=[
        pltpu.VMEM((_G, _D), jnp.float32),
        pltpu.VMEM((_G, _D), jnp.float32),
    ],
)


def kernel(x, edge_index, batch, W0, b0, Ws, bs, gammas, betas, Wf, bf):
    padv = _N + jnp.arange(_NCHK * _CH - _E, dtype=jnp.int32) % (_NP - _N)
    src2 = jnp.concatenate([edge_index[0], padv]).reshape(_NCHK, _CH)
    dst2 = jnp.concatenate([edge_index[1], padv]).reshape(_NCHK, _CH)
    xp = jnp.zeros((_NP, _D), jnp.float32).at[:_N].set(x)
    batch_p = jnp.full((_NP, 1), _G, jnp.int32).at[:_N, 0].set(batch)
    zeros_row = jnp.zeros((_CH, _D), jnp.float32)
    ones_row = jnp.ones((_CH, _D), jnp.float32)

    deg = _sc_deg(dst2, ones_row, zeros_row)
    dis, hw = _tc_pre(xp, deg, W0, b0[None, :], Ws[0])

    for i in range(_L - 1):
        accs = _sc_agg(src2, dst2, hw, zeros_row)
        hw = _tc_mid(accs, hw, dis, bs[i][None, :], gammas[i][None, :],
                     betas[i][None, :], Ws[i + 1])
    accs = _sc_agg(src2, dst2, hw, zeros_row)
    i = _L - 1
    return _tc_last(accs, hw, dis, bs[i][None, :], gammas[i][None, :],
                    betas[i][None, :], batch_p, Wf, bf[None, :])


# speedup vs baseline: 4.5037x; 1.0002x over previous
import functools

import jax
import jax.numpy as jnp
from jax import lax
from jax.experimental import pallas as pl
from jax.experimental.pallas import tpu as pltpu
from jax.experimental.pallas import tpu_sc as plsc

_N = 10000
_E = 320000
_D = 128
_L = 8
_G = 16

_NP = 10240
_NSC = 2
_NT = 16
_CH = 128
_NCHK = 2560
_CPT = _NCHK // (_NSC * _NT)
_RPT = _NP // _NT
_BN = 1024

_sc_mesh = plsc.VectorSubcoreMesh(core_axis_name="c", subcore_axis_name="s")


_BW = 40


@functools.partial(
    pl.kernel,
    out_type=jax.ShapeDtypeStruct((_NSC, _NP, _D), jnp.float32),
    mesh=_sc_mesh,
    scratch_types=[
        pltpu.VMEM((_BW, _CH), jnp.int32),
        pltpu.VMEM((_BW, _CH), jnp.int32),
        pltpu.VMEM((_CH, _D), jnp.float32),
        pltpu.VMEM((_CH, _D), jnp.float32),
        pltpu.VMEM_SHARED((_NP, _D), jnp.float32),
        pltpu.SemaphoreType.DMA,
        pltpu.SemaphoreType.DMA,
        pltpu.SemaphoreType.DMA,
        pltpu.SemaphoreType.DMA,
    ],
)
def _sc_agg(src_hbm, dst_hbm, hw_hbm, zeros_hbm, out_hbm, srcv, dstv,
            b0, b1, acc, g0, g1, s0, s1):
    c = lax.axis_index("c")
    s = lax.axis_index("s")
    wid = c * _NT + s
    bufs = (b0, b1)
    gsems = (g0, g1)
    ssems = (s0, s1)
    pltpu.sync_copy(zeros_hbm, b0)
    row0 = s * _RPT
    zds = [pltpu.async_copy(b0, acc.at[pl.ds(row0 + j * _CH, _CH)], g0)
           for j in range(_RPT // _CH)]
    for d in zds:
        d.wait()
    plsc.subcore_barrier()
    base = wid * _CPT

    def body(t, carry):
        pltpu.sync_copy(src_hbm.at[pl.ds(base + t * _BW, _BW)], srcv)
        pltpu.sync_copy(dst_hbm.at[pl.ds(base + t * _BW, _BW)], dstv)
        gds = [None, None]
        sds = [None, None]
        gds[0] = pltpu.async_copy(hw_hbm.at[srcv.at[0]], bufs[0], gsems[0])
        for j in range(_BW):
            bj = j % 2
            if j + 1 < _BW:
                bn = (j + 1) % 2
                if sds[bn] is not None:
                    sds[bn].wait()
                    sds[bn] = None
                gds[bn] = pltpu.async_copy(hw_hbm.at[srcv.at[j + 1]],
                                           bufs[bn], gsems[bn])
            gds[bj].wait()
            sds[bj] = pltpu.async_copy(bufs[bj], acc.at[dstv.at[j]],
                                       ssems[bj], add=True)
        for bj in range(2):
            if sds[bj] is not None:
                sds[bj].wait()
        return carry

    lax.fori_loop(0, _CPT // _BW, body, 0)
    plsc.subcore_barrier()
    rds = [None, None]
    wds = [None, None]
    nf = _RPT // _CH
    rds[0] = pltpu.async_copy(acc.at[pl.ds(row0, _CH)], bufs[0], gsems[0])
    for j in range(nf):
        bj = j % 2
        if j + 1 < nf:
            bn = (j + 1) % 2
            if wds[bn] is not None:
                wds[bn].wait()
            rds[bn] = pltpu.async_copy(
                acc.at[pl.ds(row0 + (j + 1) * _CH, _CH)], bufs[bn],
                gsems[bn])
        rds[bj].wait()
        wds[bj] = pltpu.async_copy(bufs[bj],
                                   out_hbm.at[c, pl.ds(row0 + j * _CH, _CH)],
                                   ssems[bj])
    for bj in range(2):
        if wds[bj] is not None:
            wds[bj].wait()


_CPTD = _NCHK // (_NSC * _NT)


@functools.partial(
    pl.kernel,
    out_type=jax.ShapeDtypeStruct((_NSC, _NP, _D), jnp.float32),
    mesh=_sc_mesh,
    scratch_types=[
        pltpu.VMEM((_BW, _CH), jnp.int32),
        pltpu.VMEM((_CH, _D), jnp.float32),
        pltpu.VMEM((_CH, _D), jnp.float32),
        pltpu.VMEM_SHARED((_NP, _D), jnp.float32),
        pltpu.SemaphoreType.DMA,
        pltpu.SemaphoreType.DMA,
        pltpu.SemaphoreType.DMA,
        pltpu.SemaphoreType.DMA,
        pltpu.SemaphoreType.DMA,
    ],
)
def _sc_deg(dst_hbm, ones_hbm, zeros_hbm, out_hbm, dstv, onesv, stage, acc,
            ssem, r0, r1, w0, w1):
    rsems = (r0, r1)
    wsems = (w0, w1)
    c = lax.axis_index("c")
    s = lax.axis_index("s")
    wid = c * _NT + s
    pltpu.sync_copy(ones_hbm, onesv)
    pltpu.sync_copy(zeros_hbm, stage)
    row0 = s * _RPT
    zds = [pltpu.async_copy(stage, acc.at[pl.ds(row0 + j * _CH, _CH)], ssem)
           for j in range(_RPT // _CH)]
    for d in zds:
        d.wait()
    plsc.subcore_barrier()
    base = wid * _CPTD

    def body(t, carry):
        pltpu.sync_copy(dst_hbm.at[pl.ds(base + t * _BW, _BW)], dstv)
        sds = [pltpu.async_copy(onesv, acc.at[dstv.at[j]], ssem, add=True)
               for j in range(_BW)]
        for d in sds:
            d.wait()
        return carry

    lax.fori_loop(0, _CPTD // _BW, body, 0)
    plsc.subcore_barrier()
    fbufs = (stage, onesv)
    rds = [None, None]
    wds = [None, None]
    nf = _RPT // _CH
    rds[0] = pltpu.async_copy(acc.at[pl.ds(row0, _CH)], fbufs[0], rsems[0])
    for j in range(nf):
        bj = j % 2
        if j + 1 < nf:
            bn = (j + 1) % 2
            if wds[bn] is not None:
                wds[bn].wait()
            rds[bn] = pltpu.async_copy(
                acc.at[pl.ds(row0 + (j + 1) * _CH, _CH)], fbufs[bn],
                rsems[bn])
        rds[bj].wait()
        wds[bj] = pltpu.async_copy(fbufs[bj],
                                   out_hbm.at[c, pl.ds(row0 + j * _CH, _CH)],
                                   wsems[bj])
    for bj in range(2):
        if wds[bj] is not None:
            wds[bj].wait()


def _tc_pre_body(x_ref, deg_ref, W0_ref, b0_ref, Ws0_ref, dis_ref, hw_ref):
    counts = deg_ref[0][:, 0:1] + deg_ref[1][:, 0:1]
    dis = lax.rsqrt(counts + 1.0)
    h0 = jnp.dot(x_ref[...], W0_ref[...], preferred_element_type=jnp.float32)
    h0 = h0 + b0_ref[...]
    hw = jnp.dot(h0, Ws0_ref[...], preferred_element_type=jnp.float32) * dis
    dis_ref[...] = dis
    hw_ref[...] = hw


_tc_pre = pl.pallas_call(
    _tc_pre_body,
    grid=(_NP // _BN,),
    in_specs=[
        pl.BlockSpec((_BN, _D), lambda i: (i, 0)),
        pl.BlockSpec((_NSC, _BN, _D), lambda i: (0, i, 0)),
        pl.BlockSpec((_D, _D), lambda i: (0, 0)),
        pl.BlockSpec((1, _D), lambda i: (0, 0)),
        pl.BlockSpec((_D, _D), lambda i: (0, 0)),
    ],
    out_specs=[
        pl.BlockSpec((_BN, 1), lambda i: (i, 0)),
        pl.BlockSpec((_BN, _D), lambda i: (i, 0)),
    ],
    out_shape=[
        jax.ShapeDtypeStruct((_NP, 1), jnp.float32),
        jax.ShapeDtypeStruct((_NP, _D), jnp.float32),
    ],
)


def _tc_mid_body(acc_ref, hw_ref, dis_ref, b_ref, g_ref, be_ref, Wn_ref,
                 hw_o):
    dis = dis_ref[...]
    t = (acc_ref[0] + acc_ref[1] + hw_ref[...]) * dis + b_ref[...]
    mu = jnp.mean(t, axis=-1, keepdims=True)
    d = t - mu
    var = jnp.mean(d * d, axis=-1, keepdims=True)
    tn = d * lax.rsqrt(var + 1e-5) * g_ref[...] + be_ref[...]
    h = jnp.maximum(tn, 0.0)
    hw_o[...] = (jnp.dot(h, Wn_ref[...], preferred_element_type=jnp.float32)
                 * dis)


_tc_mid = pl.pallas_call(
    _tc_mid_body,
    grid=(_NP // _BN,),
    in_specs=[
        pl.BlockSpec((_NSC, _BN, _D), lambda i: (0, i, 0)),
        pl.BlockSpec((_BN, _D), lambda i: (i, 0)),
        pl.BlockSpec((_BN, 1), lambda i: (i, 0)),
        pl.BlockSpec((1, _D), lambda i: (0, 0)),
        pl.BlockSpec((1, _D), lambda i: (0, 0)),
        pl.BlockSpec((1, _D), lambda i: (0, 0)),
        pl.BlockSpec((_D, _D), lambda i: (0, 0)),
    ],
    out_specs=pl.BlockSpec((_BN, _D), lambda i: (i, 0)),
    out_shape=jax.ShapeDtypeStruct((_NP, _D), jnp.float32),
)


def _tc_last_body(acc_ref, hw_ref, dis_ref, b_ref, g_ref, be_ref, batch_ref,
                  Wf_ref, bf_ref, out_ref, pool_ref, cnt_ref):
    i = pl.program_id(0)

    @pl.when(i == 0)
    def _():
        pool_ref[...] = jnp.zeros_like(pool_ref)
        cnt_ref[...] = jnp.zeros_like(cnt_ref)

    dis = dis_ref[...]
    t = (acc_ref[0] + acc_ref[1] + hw_ref[...]) * dis + b_ref[...]
    mu = jnp.mean(t, axis=-1, keepdims=True)
    d = t - mu
    var = jnp.mean(d * d, axis=-1, keepdims=True)
    tn = d * lax.rsqrt(var + 1e-5) * g_ref[...] + be_ref[...]
    h = jnp.maximum(tn, 0.0)
    onehot = (batch_ref[...] ==
              lax.broadcasted_iota(jnp.int32, (1, _G), 1)).astype(jnp.float32)
    pool_ref[...] += lax.dot_general(
        onehot, h, (((0,), (0,)), ((), ())),
        preferred_element_type=jnp.float32)
    cnt_ref[...] += lax.dot_general(
        onehot, jnp.ones((_BN, _D), jnp.float32), (((0,), (0,)), ((), ())),
        preferred_element_type=jnp.float32)

    @pl.when(i == pl.num_programs(0) - 1)
    def _():
        pooled = pool_ref[...] / jnp.maximum(cnt_ref[...], 1.0)
        out_ref[...] = (jnp.dot(pooled, Wf_ref[...],
                                preferred_element_type=jnp.float32)
                        + bf_ref[...])


_tc_last = pl.pallas_call(
    _tc_last_body,
    grid=(_NP // _BN,),
    in_specs=[
        pl.BlockSpec((_NSC, _BN, _D), lambda i: (0, i, 0)),
        pl.BlockSpec((_BN, _D), lambda i: (i, 0)),
        pl.BlockSpec((_BN, 1), lambda i: (i, 0)),
        pl.BlockSpec((1, _D), lambda i: (0, 0)),
        pl.BlockSpec((1, _D), lambda i: (0, 0)),
        pl.BlockSpec((1, _D), lambda i: (0, 0)),
        pl.BlockSpec((_BN, 1), lambda i: (i, 0)),
        pl.BlockSpec((_D, _D), lambda i: (0, 0)),
        pl.BlockSpec((1, _D), lambda i: (0, 0)),
    ],
    out_specs=pl.BlockSpec((_G, _D), lambda i: (0, 0)),
    out_shape=jax.ShapeDtypeStruct((_G, _D), jnp.float32),
    scratch_shapes=[
        pltpu.VMEM((_G, _D), jnp.float32),
        pltpu.VMEM((_G, _D), jnp.float32),
    ],
)


def kernel(x, edge_index, batch, W0, b0, Ws, bs, gammas, betas, Wf, bf):
    padv = _N + jnp.arange(_NCHK * _CH - _E, dtype=jnp.int32) % (_NP - _N)
    src2 = jnp.concatenate([edge_index[0], padv]).reshape(_NCHK, _CH)
    dst2 = jnp.concatenate([edge_index[1], padv]).reshape(_NCHK, _CH)
    xp = jnp.zeros((_NP, _D), jnp.float32).at[:_N].set(x)
    batch_p = jnp.full((_NP, 1), _G, jnp.int32).at[:_N, 0].set(batch)
    zeros_row = jnp.zeros((_CH, _D), jnp.float32)
    ones_row = jnp.ones((_CH, _D), jnp.float32)

    deg = _sc_deg(dst2, ones_row, zeros_row)
    dis, hw = _tc_pre(xp, deg, W0, b0[None, :], Ws[0])

    for i in range(_L - 1):
        accs = _sc_agg(src2, dst2, hw, zeros_row)
        hw = _tc_mid(accs, hw, dis, bs[i][None, :], gammas[i][None, :],
                     betas[i][None, :], Ws[i + 1])
    accs = _sc_agg(src2, dst2, hw, zeros_row)
    i = _L - 1
    return _tc_last(accs, hw, dis, bs[i][None, :], gammas[i][None, :],
                    betas[i][None, :], batch_p, Wf, bf[None, :])
